# baseline (speedup 1.0000x reference)
import functools

import jax
import jax.numpy as jnp
from jax import lax
from jax.experimental import pallas as pl
from jax.experimental.pallas import tpu as pltpu
from jax.experimental.pallas import tpu_sc as plsc

N_NODES = 10000
D_FEAT = 128
F_HID = 16
N_EDGES = 320000

NC = 2
NS = 16
NW = NC * NS
CH = 128
NCH = 2560
PER_W = NCH // NW
C0_N = 80
C1_N = 80
C0_TOT = NS * C0_N
E_PAD = NCH * CH
PAD_DST = N_NODES
NP = 10240
ROWS_PER_TILE = NP // NS

_sc_mesh = plsc.VectorSubcoreMesh(core_axis_name="c", subcore_axis_name="s")
_sc_params = pltpu.CompilerParams(use_tc_tiling_on_sc=False)


def _worker_id():
    return lax.axis_index("c") * NS + lax.axis_index("s")


@functools.partial(
    pl.kernel,
    out_type=jax.ShapeDtypeStruct((NC, NP), jnp.float32),
    mesh=_sc_mesh,
    compiler_params=_sc_params,
    scratch_types=[
        pltpu.VMEM_SHARED((NP,), jnp.float32),
        pltpu.VMEM((PER_W, CH), jnp.int32),
        pltpu.VMEM((CH,), jnp.float32),
        pltpu.VMEM((ROWS_PER_TILE,), jnp.float32),
        pltpu.SemaphoreType.DMA,
    ],
)
def _sc_degree(dst_hbm, deg_out, deg_sh, dst_v, ones_v, z_v, dsem):
    c = lax.axis_index("c")
    s = lax.axis_index("s")
    w = _worker_id()

    def fill_z(i, _):
        z_v[pl.ds(i * 16, 16)] = jnp.zeros((16,), jnp.float32)
        return _

    lax.fori_loop(0, ROWS_PER_TILE // 16, fill_z, None)

    def fill_ones(i, _):
        ones_v[pl.ds(i * 16, 16)] = jnp.ones((16,), jnp.float32)
        return _

    lax.fori_loop(0, CH // 16, fill_ones, None)

    pltpu.sync_copy(z_v, deg_sh.at[pl.ds(s * ROWS_PER_TILE, ROWS_PER_TILE)])
    plsc.subcore_barrier()

    pltpu.sync_copy(dst_hbm.at[pl.ds(w * PER_W, PER_W)], dst_v)

    def body(j, _):
        pltpu.make_async_copy(
            ones_v, deg_sh.at[dst_v.at[j]], dsem).start(add=True)
        return _

    lax.fori_loop(0, PER_W, body, None)

    def drain(j, _):
        pltpu.make_async_copy(ones_v, deg_sh.at[dst_v.at[0]], dsem).wait()
        return _

    lax.fori_loop(0, PER_W, drain, None)
    plsc.subcore_barrier()
    pltpu.sync_copy(
        deg_sh.at[pl.ds(s * ROWS_PER_TILE, ROWS_PER_TILE)],
        deg_out.at[c, pl.ds(s * ROWS_PER_TILE, ROWS_PER_TILE)],
    )


@functools.partial(
    pl.kernel,
    out_type=jax.ShapeDtypeStruct((NC, NP, F_HID), jnp.float32),
    mesh=_sc_mesh,
    compiler_params=_sc_params,
    scratch_types=[
        pltpu.VMEM_SHARED((NP, F_HID), jnp.float32),
        pltpu.VMEM((C0_N, CH), jnp.int32),
        pltpu.VMEM((C0_N, CH), jnp.int32),
        pltpu.VMEM((8, CH, F_HID), jnp.float32),
        pltpu.VMEM((ROWS_PER_TILE, F_HID), jnp.float32),
        pltpu.SemaphoreType.DMA((8,)),
        pltpu.SemaphoreType.DMA((8,)),
    ],
)
def _sc_propagate(hs_hbm, src_hbm, dst_hbm, acc_out,
                  acc_sh, src_v, dst_v, rows_v, z_v, gsems, ssems):
    c = lax.axis_index("c")
    s = lax.axis_index("s")

    def fill_z(i, _):
        z_v[i, :] = jnp.zeros((F_HID,), jnp.float32)
        return _

    lax.fori_loop(0, ROWS_PER_TILE, fill_z, None)
    pltpu.sync_copy(z_v, acc_sh.at[pl.ds(s * ROWS_PER_TILE, ROWS_PER_TILE)])
    plsc.subcore_barrier()

    def edge_pipeline(base, n):
        pltpu.sync_copy(src_hbm.at[pl.ds(base, n)], src_v.at[pl.ds(0, n)])
        pltpu.sync_copy(dst_hbm.at[pl.ds(base, n)], dst_v.at[pl.ds(0, n)])

        def gather(j, b):
            pltpu.make_async_copy(
                hs_hbm.at[src_v.at[j]], rows_v.at[b], gsems.at[b]).start()

        def scat(j, b):
            return pltpu.make_async_copy(
                rows_v.at[b], acc_sh.at[dst_v.at[j]], ssems.at[b])

        for b in range(4):
            gather(b, b)

        def group(i, _):
            j0 = 8 * i
            for b in range(8):
                j = j0 + b
                pltpu.make_async_copy(
                    hs_hbm.at[src_v.at[j]], rows_v.at[b], gsems.at[b]).wait()
                scat(j, b).start(add=True)
                pb = (b + 4) % 8

                @pl.when(j >= 4)
                def _free():
                    scat(j - 4, pb).wait()

                @pl.when(j + 4 < n)
                def _next():
                    gather(j + 4, pb)

            return _

        lax.fori_loop(0, n // 8, group, None)
        for b in range(4, 8):
            scat(n - 8 + b, b).wait()

    @pl.when(c == 0)
    def _core0():
        edge_pipeline(s * C0_N, C0_N)

    @pl.when(c == 1)
    def _core1():
        edge_pipeline(C0_TOT + s * C1_N, C1_N)

    plsc.subcore_barrier()
    pltpu.sync_copy(
        acc_sh.at[pl.ds(s * ROWS_PER_TILE, ROWS_PER_TILE)],
        acc_out.at[c, pl.ds(s * ROWS_PER_TILE, ROWS_PER_TILE)],
    )


def _bf16(t):
    return t.astype(jnp.bfloat16).astype(jnp.float32)


def _prep_body(ei_ref, src_out, dst_out):
    e_rows = N_EDGES // CH
    src_out[0:e_rows, :] = ei_ref[0, :].reshape(e_rows, CH)
    src_out[e_rows:NCH, :] = jnp.zeros((NCH - e_rows, CH), jnp.int32)
    dst_out[0:e_rows, :] = ei_ref[1, :].reshape(e_rows, CH)
    flat = (lax.broadcasted_iota(jnp.int32, (NCH - e_rows, CH), 0) * CH
            + lax.broadcasted_iota(jnp.int32, (NCH - e_rows, CH), 1))
    dst_out[e_rows:NCH, :] = N_NODES + flat % (NP - N_NODES)


_prep = pl.pallas_call(
    _prep_body,
    out_shape=(
        jax.ShapeDtypeStruct((NCH, CH), jnp.int32),
        jax.ShapeDtypeStruct((NCH, CH), jnp.int32),
    ),
)


def _tcmm_body(x_ref, w1_ref, h_ref):
    h_ref[...] = jnp.dot(_bf16(x_ref[...]), _bf16(w1_ref[...]),
                         preferred_element_type=jnp.float32,
                         precision=lax.Precision.HIGHEST)


_tcmm = pl.pallas_call(
    _tcmm_body,
    out_shape=jax.ShapeDtypeStruct((N_NODES, F_HID), jnp.float32),
)


def _tc1_body(deg_p_ref, h_ref, dis_ref, hs1_ref):
    deg = deg_p_ref[0, :] + deg_p_ref[1, :] + 1.0
    dis = lax.rsqrt(deg)
    dis = dis * (1.5 - 0.5 * deg * dis * dis)
    dis_ref[...] = dis
    hs1_ref[...] = h_ref[...] * dis[:N_NODES, None]


_tc1 = pl.pallas_call(
    _tc1_body,
    out_shape=(
        jax.ShapeDtypeStruct((NP,), jnp.float32),
        jax.ShapeDtypeStruct((N_NODES, F_HID), jnp.float32),
    ),
)


def _tc2_body(acc_p_ref, hs1_ref, dis_ref, w2_ref, b1_ref, hs2_ref):
    acc = acc_p_ref[0, :N_NODES, :] + acc_p_ref[1, :N_NODES, :]
    dis = dis_ref[...][:N_NODES, None]
    h1 = jnp.maximum(dis * (acc + hs1_ref[...]) + b1_ref[...], 0.0)
    hs2_ref[...] = dis * jnp.dot(_bf16(h1), _bf16(w2_ref[...]),
                                 preferred_element_type=jnp.float32,
                         precision=lax.Precision.HIGHEST)


_tc2 = pl.pallas_call(
    _tc2_body,
    out_shape=jax.ShapeDtypeStruct((N_NODES, F_HID), jnp.float32),
)


def _tc3_body(acc_p_ref, hs2_ref, dis_ref, b2_ref, wc_ref, bc_ref,
              wfa_ref, wfb_ref, bfa_ref, bfb_ref,
              critic_ref, na_ref, nb_ref):
    acc = acc_p_ref[0, :N_NODES, :] + acc_p_ref[1, :N_NODES, :]
    dis = dis_ref[...][:N_NODES, None]
    h2 = jnp.maximum(dis * (acc + hs2_ref[...]) + b2_ref[...], 0.0)
    wc = _bf16(wc_ref[...][:, 0])
    critic_ref[...] = jnp.sum(_bf16(h2) * wc[None, :], axis=1,
                              keepdims=True) + bc_ref[0]
    pooled = jnp.mean(h2, axis=0, keepdims=True)
    oa = jnp.dot(_bf16(pooled), _bf16(wfa_ref[...]),
                 preferred_element_type=jnp.float32,
                         precision=lax.Precision.HIGHEST) + bfa_ref[...][None, :]
    ob = jnp.dot(_bf16(pooled), _bf16(wfb_ref[...]),
                 preferred_element_type=jnp.float32,
                         precision=lax.Precision.HIGHEST) + bfb_ref[...][None, :]
    m = jnp.maximum(oa, ob)
    ea = jnp.exp(oa - m)
    eb = jnp.exp(ob - m)
    inv = 127.0 / (ea + eb)
    na_ref[...] = jnp.round(ea * inv).astype(jnp.int32)
    nb_ref[...] = jnp.round(eb * inv).astype(jnp.int32)


_tc3 = pl.pallas_call(
    _tc3_body,
    out_shape=(
        jax.ShapeDtypeStruct((N_NODES, 1), jnp.float32),
        jax.ShapeDtypeStruct((1, 64), jnp.int32),
        jax.ShapeDtypeStruct((1, 64), jnp.int32),
    ),
)


def kernel(x, edge_index, W1, b1, W2, b2, Wfc, bfc, Wc, bc):
    src2d, dst2d = _prep(edge_index.astype(jnp.int32))

    deg_p = _sc_degree(dst2d)
    h = _tcmm(x, W1)
    dis, hs1 = _tc1(deg_p, h)
    acc1 = _sc_propagate(hs1, src2d, dst2d)
    hs2 = _tc2(acc1, hs1, dis, W2, b1)
    acc2 = _sc_propagate(hs2, src2d, dst2d)
    critic, na, nb = _tc3(acc2, hs2, dis, b2, Wc, bc,
                          Wfc[:, 0::2], Wfc[:, 1::2], bfc[0::2], bfc[1::2])
    nodes_chosen = jnp.stack([na[0], nb[0]], axis=1)
    return (nodes_chosen, critic)


# speedup vs baseline: 1.0559x; 1.0559x over previous
import functools

import jax
import jax.numpy as jnp
from jax import lax
from jax.experimental import pallas as pl
from jax.experimental.pallas import tpu as pltpu
from jax.experimental.pallas import tpu_sc as plsc

N_NODES = 10000
D_FEAT = 128
F_HID = 16
N_EDGES = 320000

NC = 2
NS = 16
NW = NC * NS
CH = 128
NCH = 2560
PER_W = NCH // NW
C0_N = 128
C1_N = 32
C0_TOT = NS * C0_N
E_PAD = NCH * CH
PAD_DST = N_NODES
NP = 10240
ROWS_PER_TILE = NP // NS

_sc_mesh = plsc.VectorSubcoreMesh(core_axis_name="c", subcore_axis_name="s")
_sc_params = pltpu.CompilerParams(use_tc_tiling_on_sc=False)


def _worker_id():
    return lax.axis_index("c") * NS + lax.axis_index("s")


@functools.partial(
    pl.kernel,
    out_type=jax.ShapeDtypeStruct((NC, NP), jnp.float32),
    mesh=_sc_mesh,
    compiler_params=_sc_params,
    scratch_types=[
        pltpu.VMEM_SHARED((NP,), jnp.float32),
        pltpu.VMEM((PER_W, CH), jnp.int32),
        pltpu.VMEM((CH,), jnp.float32),
        pltpu.VMEM((ROWS_PER_TILE,), jnp.float32),
        pltpu.SemaphoreType.DMA,
    ],
)
def _sc_degree(dst_hbm, deg_out, deg_sh, dst_v, ones_v, z_v, dsem):
    c = lax.axis_index("c")
    s = lax.axis_index("s")
    w = _worker_id()

    def fill_z(i, _):
        z_v[pl.ds(i * 16, 16)] = jnp.zeros((16,), jnp.float32)
        return _

    lax.fori_loop(0, ROWS_PER_TILE // 16, fill_z, None)

    def fill_ones(i, _):
        ones_v[pl.ds(i * 16, 16)] = jnp.ones((16,), jnp.float32)
        return _

    lax.fori_loop(0, CH // 16, fill_ones, None)

    pltpu.sync_copy(z_v, deg_sh.at[pl.ds(s * ROWS_PER_TILE, ROWS_PER_TILE)])
    plsc.subcore_barrier()

    pltpu.sync_copy(dst_hbm.at[pl.ds(w * PER_W, PER_W)], dst_v)

    def body(j, _):
        pltpu.make_async_copy(
            ones_v, deg_sh.at[dst_v.at[j]], dsem).start(add=True)
        return _

    lax.fori_loop(0, PER_W, body, None)

    def drain(j, _):
        pltpu.make_async_copy(ones_v, deg_sh.at[dst_v.at[0]], dsem).wait()
        return _

    lax.fori_loop(0, PER_W, drain, None)
    plsc.subcore_barrier()
    pltpu.sync_copy(
        deg_sh.at[pl.ds(s * ROWS_PER_TILE, ROWS_PER_TILE)],
        deg_out.at[c, pl.ds(s * ROWS_PER_TILE, ROWS_PER_TILE)],
    )


@functools.partial(
    pl.kernel,
    out_type=jax.ShapeDtypeStruct((NC, NP, F_HID), jnp.float32),
    mesh=_sc_mesh,
    compiler_params=_sc_params,
    scratch_types=[
        pltpu.VMEM_SHARED((NP, F_HID), jnp.float32),
        pltpu.VMEM((C0_N, CH), jnp.int32),
        pltpu.VMEM((C0_N, CH), jnp.int32),
        pltpu.VMEM((8, CH, F_HID), jnp.float32),
        pltpu.VMEM((ROWS_PER_TILE, F_HID), jnp.float32),
        pltpu.SemaphoreType.DMA((8,)),
        pltpu.SemaphoreType.DMA((8,)),
    ],
)
def _sc_propagate(hs_hbm, src_hbm, dst_hbm, acc_out,
                  acc_sh, src_v, dst_v, rows_v, z_v, gsems, ssems):
    c = lax.axis_index("c")
    s = lax.axis_index("s")

    def fill_z(i, _):
        z_v[i, :] = jnp.zeros((F_HID,), jnp.float32)
        return _

    lax.fori_loop(0, ROWS_PER_TILE, fill_z, None)
    pltpu.sync_copy(z_v, acc_sh.at[pl.ds(s * ROWS_PER_TILE, ROWS_PER_TILE)])
    plsc.subcore_barrier()

    def edge_pipeline(base, n):
        pltpu.sync_copy(src_hbm.at[pl.ds(base, n)], src_v.at[pl.ds(0, n)])
        pltpu.sync_copy(dst_hbm.at[pl.ds(base, n)], dst_v.at[pl.ds(0, n)])

        def gather(j, b):
            pltpu.make_async_copy(
                hs_hbm.at[src_v.at[j]], rows_v.at[b], gsems.at[b]).start()

        def scat(j, b):
            return pltpu.make_async_copy(
                rows_v.at[b], acc_sh.at[dst_v.at[j]], ssems.at[b])

        for b in range(4):
            gather(b, b)

        def group(i, _):
            j0 = 8 * i
            for b in range(8):
                j = j0 + b
                pltpu.make_async_copy(
                    hs_hbm.at[src_v.at[j]], rows_v.at[b], gsems.at[b]).wait()
                scat(j, b).start(add=True)
                pb = (b + 4) % 8

                @pl.when(j >= 4)
                def _free():
                    scat(j - 4, pb).wait()

                @pl.when(j + 4 < n)
                def _next():
                    gather(j + 4, pb)

            return _

        lax.fori_loop(0, n // 8, group, None)
        for b in range(4, 8):
            scat(n - 8 + b, b).wait()

    @pl.when(c == 0)
    def _core0():
        edge_pipeline(s * C0_N, C0_N)

    @pl.when(c == 1)
    def _core1():
        edge_pipeline(C0_TOT + s * C1_N, C1_N)

    plsc.subcore_barrier()
    pltpu.sync_copy(
        acc_sh.at[pl.ds(s * ROWS_PER_TILE, ROWS_PER_TILE)],
        acc_out.at[c, pl.ds(s * ROWS_PER_TILE, ROWS_PER_TILE)],
    )


def _bf16(t):
    return t.astype(jnp.bfloat16).astype(jnp.float32)


def _prep_body(ei_ref, src_out, dst_out):
    e_rows = N_EDGES // CH
    src_out[0:e_rows, :] = ei_ref[0, :].reshape(e_rows, CH)
    src_out[e_rows:NCH, :] = jnp.zeros((NCH - e_rows, CH), jnp.int32)
    dst_out[0:e_rows, :] = ei_ref[1, :].reshape(e_rows, CH)
    flat = (lax.broadcasted_iota(jnp.int32, (NCH - e_rows, CH), 0) * CH
            + lax.broadcasted_iota(jnp.int32, (NCH - e_rows, CH), 1))
    dst_out[e_rows:NCH, :] = N_NODES + flat % (NP - N_NODES)


_prep = pl.pallas_call(
    _prep_body,
    out_shape=(
        jax.ShapeDtypeStruct((NCH, CH), jnp.int32),
        jax.ShapeDtypeStruct((NCH, CH), jnp.int32),
    ),
)


def _tcmm_body(x_ref, w1_ref, h_ref):
    h_ref[...] = jnp.dot(_bf16(x_ref[...]), _bf16(w1_ref[...]),
                         preferred_element_type=jnp.float32,
                         precision=lax.Precision.HIGHEST)


_tcmm = pl.pallas_call(
    _tcmm_body,
    out_shape=jax.ShapeDtypeStruct((N_NODES, F_HID), jnp.float32),
)


def _tc1_body(deg_p_ref, h_ref, dis_ref, hs1_ref):
    deg = deg_p_ref[0, :] + deg_p_ref[1, :] + 1.0
    dis = lax.rsqrt(deg)
    dis = dis * (1.5 - 0.5 * deg * dis * dis)
    dis_ref[...] = dis
    hs1_ref[...] = h_ref[...] * dis[:N_NODES, None]


_tc1 = pl.pallas_call(
    _tc1_body,
    out_shape=(
        jax.ShapeDtypeStruct((NP,), jnp.float32),
        jax.ShapeDtypeStruct((N_NODES, F_HID), jnp.float32),
    ),
)


def _tc2_body(acc_p_ref, hs1_ref, dis_ref, w2_ref, b1_ref, hs2_ref):
    acc = acc_p_ref[0, :N_NODES, :] + acc_p_ref[1, :N_NODES, :]
    dis = dis_ref[...][:N_NODES, None]
    h1 = jnp.maximum(dis * (acc + hs1_ref[...]) + b1_ref[...], 0.0)
    hs2_ref[...] = dis * jnp.dot(_bf16(h1), _bf16(w2_ref[...]),
                                 preferred_element_type=jnp.float32,
                         precision=lax.Precision.HIGHEST)


_tc2 = pl.pallas_call(
    _tc2_body,
    out_shape=jax.ShapeDtypeStruct((N_NODES, F_HID), jnp.float32),
)


def _tc3_body(acc_p_ref, hs2_ref, dis_ref, b2_ref, wc_ref, bc_ref,
              wfa_ref, wfb_ref, bfa_ref, bfb_ref,
              critic_ref, na_ref, nb_ref):
    acc = acc_p_ref[0, :N_NODES, :] + acc_p_ref[1, :N_NODES, :]
    dis = dis_ref[...][:N_NODES, None]
    h2 = jnp.maximum(dis * (acc + hs2_ref[...]) + b2_ref[...], 0.0)
    wc = _bf16(wc_ref[...][:, 0])
    critic_ref[...] = jnp.sum(_bf16(h2) * wc[None, :], axis=1,
                              keepdims=True) + bc_ref[0]
    pooled = jnp.mean(h2, axis=0, keepdims=True)
    oa = jnp.dot(_bf16(pooled), _bf16(wfa_ref[...]),
                 preferred_element_type=jnp.float32,
                         precision=lax.Precision.HIGHEST) + bfa_ref[...][None, :]
    ob = jnp.dot(_bf16(pooled), _bf16(wfb_ref[...]),
                 preferred_element_type=jnp.float32,
                         precision=lax.Precision.HIGHEST) + bfb_ref[...][None, :]
    m = jnp.maximum(oa, ob)
    ea = jnp.exp(oa - m)
    eb = jnp.exp(ob - m)
    inv = 127.0 / (ea + eb)
    na_ref[...] = jnp.round(ea * inv).astype(jnp.int32)
    nb_ref[...] = jnp.round(eb * inv).astype(jnp.int32)


_tc3 = pl.pallas_call(
    _tc3_body,
    out_shape=(
        jax.ShapeDtypeStruct((N_NODES, 1), jnp.float32),
        jax.ShapeDtypeStruct((1, 64), jnp.int32),
        jax.ShapeDtypeStruct((1, 64), jnp.int32),
    ),
)


def kernel(x, edge_index, W1, b1, W2, b2, Wfc, bfc, Wc, bc):
    src2d, dst2d = _prep(edge_index.astype(jnp.int32))

    deg_p = _sc_degree(dst2d)
    h = _tcmm(x, W1)
    dis, hs1 = _tc1(deg_p, h)
    acc1 = _sc_propagate(hs1, src2d, dst2d)
    hs2 = _tc2(acc1, hs1, dis, W2, b1)
    acc2 = _sc_propagate(hs2, src2d, dst2d)
    critic, na, nb = _tc3(acc2, hs2, dis, b2, Wc, bc,
                          Wfc[:, 0::2], Wfc[:, 1::2], bfc[0::2], bfc[1::2])
    nodes_chosen = jnp.stack([na[0], nb[0]], axis=1)
    return (nodes_chosen, critic)
